# baseline (speedup 1.0000x reference)
import jax
import jax.numpy as jnp
from jax import lax
from jax.experimental import pallas as pl
from jax.experimental.pallas import tpu as pltpu
from jax.experimental.pallas import tpu_sc as plsc

BATCH = 16384
EMBED_DIM = 32
GLOBAL_MEAN = 3.5
NUM_ROWS = 1000000

try:
    _info = plsc.get_sparse_core_info()
    NUM_CORES, NUM_SUBCORES, LANES = (
        _info.num_cores, _info.num_subcores, _info.num_lanes)
except Exception:
    NUM_CORES, NUM_SUBCORES, LANES = 2, 16, 16

NUM_WORKERS = NUM_CORES * NUM_SUBCORES
B_PER_W = BATCH // NUM_WORKERS
GROUPS = B_PER_W // LANES


def _body(x_hbm, p_hbm, q_hbm, bu_hbm, bi_hbm, out_hbm,
          uid_v, iid_v, p_rows, q_rows, bu_f, bi_f, out_v, sem):
    wid = lax.axis_index("s") * NUM_CORES + lax.axis_index("c")
    base = wid * B_PER_W

    pltpu.sync_copy(x_hbm.at[0, pl.ds(base, B_PER_W)], uid_v)
    pltpu.sync_copy(x_hbm.at[1, pl.ds(base, B_PER_W)], iid_v)

    def fix_ids(i, _):
        u = uid_v[pl.ds(i * LANES, LANES)] - 1
        uid_v[pl.ds(i * LANES, LANES)] = jnp.where(u < 0, u + NUM_ROWS, u)
        t = iid_v[pl.ds(i * LANES, LANES)] - 1
        iid_v[pl.ds(i * LANES, LANES)] = jnp.where(t < 0, t + NUM_ROWS, t)
        return _
    lax.fori_loop(0, GROUPS, fix_ids, None)

    cps = [
        pltpu.async_copy(p_hbm.at[uid_v], p_rows, sem),
        pltpu.async_copy(q_hbm.at[iid_v], q_rows, sem),
        pltpu.async_copy(bu_hbm.at[uid_v], bu_f, sem),
        pltpu.async_copy(bi_hbm.at[iid_v], bi_f, sem),
    ]
    for c in cps:
        c.wait()

    def dot_group(g, _):
        rows = g * LANES + lax.iota(jnp.int32, LANES)
        acc = (plsc.load_gather(bu_f, [rows])
               + plsc.load_gather(bi_f, [rows])
               + GLOBAL_MEAN)
        for d in range(EMBED_DIM):
            dd = jnp.full((LANES,), d, jnp.int32)
            acc += (plsc.load_gather(p_rows, [rows, dd])
                    * plsc.load_gather(q_rows, [rows, dd]))
        plsc.store_scatter(out_v, [rows], acc)
        return _
    lax.fori_loop(0, GROUPS, dot_group, None)

    pltpu.sync_copy(out_v, out_hbm.at[pl.ds(base, B_PER_W)])


@jax.jit
def kernel(x, P, Q, B_U, B_I):
    mesh = plsc.VectorSubcoreMesh(core_axis_name="c", subcore_axis_name="s")
    f = pl.kernel(
        _body,
        out_type=jax.ShapeDtypeStruct((BATCH,), jnp.float32),
        mesh=mesh,
        compiler_params=pltpu.CompilerParams(
            needs_layout_passes=False, use_tc_tiling_on_sc=False),
        scratch_types=[
            pltpu.VMEM((B_PER_W,), jnp.int32),
            pltpu.VMEM((B_PER_W,), jnp.int32),
            pltpu.VMEM((B_PER_W, EMBED_DIM), jnp.float32),
            pltpu.VMEM((B_PER_W, EMBED_DIM), jnp.float32),
            pltpu.VMEM((B_PER_W,), jnp.float32),
            pltpu.VMEM((B_PER_W,), jnp.float32),
            pltpu.VMEM((B_PER_W,), jnp.float32),
            pltpu.SemaphoreType.DMA,
        ],
    )
    return f(x, P, Q, B_U.reshape(-1), B_I.reshape(-1))
